# baseline (speedup 1.0000x reference)
import jax
import jax.numpy as jnp
from jax.experimental import pallas as pl
from jax.experimental.pallas import tpu as pltpu

_NUM_LEVELS = 4
_K = 1024
_D = 256
_B = 8192
_TILE = 256


def _vq_body(cbn_ref, x_ref, cb0_ref, cb1_ref, cb2_ref, cb3_ref,
             idx_ref, rs_ref, es_ref, zh_ref, cnt_ref, counts_scr):
    i = pl.program_id(0)
    n = pl.num_programs(0)

    @pl.when(i == 0)
    def _():
        counts_scr[...] = jnp.zeros_like(counts_scr)

    cb_refs = (cb0_ref, cb1_ref, cb2_ref, cb3_ref)
    xl = x_ref[...]
    zh = None
    for l in range(_NUM_LEVELS):
        cb = cb_refs[l][...]
        xn = jnp.sum(xl * xl, axis=1, keepdims=True)
        xc = jax.lax.dot_general(xl, cb, (((1,), (1,)), ((), ())),
                                 preferred_element_type=jnp.float32)
        d2 = (xn - 2.0 * xc) + cbn_ref[l, :][None, :]
        d = jnp.sqrt(jnp.maximum(d2, 0.0))
        m = jnp.min(d, axis=1, keepdims=True)
        jidx = jax.lax.broadcasted_iota(jnp.int32, d.shape, 1)
        idx = jnp.min(jnp.where(d == m, jidx, _K), axis=1)
        oh = (jidx == idx[:, None]).astype(jnp.float32)
        q = jax.lax.dot_general(oh, cb, (((1,), (0,)), ((), ())),
                                precision=jax.lax.Precision.HIGHEST,
                                preferred_element_type=jnp.float32)
        idx_ref[l, :] = idx
        rs_ref[:, l * _D:(l + 1) * _D] = xl
        es_ref[:, l * _D:(l + 1) * _D] = q
        counts_scr[l, :] += jnp.sum(oh, axis=0)
        zh = q if zh is None else zh + q
        xl = xl - q
    zh_ref[...] = zh

    @pl.when(i == n - 1)
    def _():
        unused = jnp.sum((counts_scr[...] == 0.0).astype(jnp.int32))
        cnt_ref[0, 0] = unused


def kernel(x, cb0, cb1, cb2, cb3):
    cbs = (cb0, cb1, cb2, cb3)
    cbn = jnp.stack([jnp.sum(cb * cb, axis=1) for cb in cbs], axis=0)

    grid = (_B // _TILE,)
    out_shapes = (
        jax.ShapeDtypeStruct((_NUM_LEVELS, _B), jnp.int32),
        jax.ShapeDtypeStruct((_B, _NUM_LEVELS * _D), jnp.float32),
        jax.ShapeDtypeStruct((_B, _NUM_LEVELS * _D), jnp.float32),
        jax.ShapeDtypeStruct((_B, _D), jnp.float32),
        jax.ShapeDtypeStruct((1, 1), jnp.int32),
    )
    in_specs = [
        pl.BlockSpec((_NUM_LEVELS, _K), lambda i: (0, 0)),
        pl.BlockSpec((_TILE, _D), lambda i: (i, 0)),
    ] + [pl.BlockSpec((_K, _D), lambda i: (0, 0))] * _NUM_LEVELS
    out_specs = [
        pl.BlockSpec((_NUM_LEVELS, _TILE), lambda i: (0, i)),
        pl.BlockSpec((_TILE, _NUM_LEVELS * _D), lambda i: (i, 0)),
        pl.BlockSpec((_TILE, _NUM_LEVELS * _D), lambda i: (i, 0)),
        pl.BlockSpec((_TILE, _D), lambda i: (i, 0)),
        pl.BlockSpec((1, 1), lambda i: (0, 0), memory_space=pltpu.SMEM),
    ]
    idx, rs, es, zh, cnt = pl.pallas_call(
        _vq_body,
        grid=grid,
        in_specs=in_specs,
        out_specs=out_specs,
        out_shape=out_shapes,
        scratch_shapes=[pltpu.VMEM((_NUM_LEVELS, _K), jnp.float32)],
    )(cbn, x, cb0, cb1, cb2, cb3)

    output = idx.T.astype(jnp.int64)
    r_s = rs.reshape(_B, _NUM_LEVELS, _D)
    e_s = es.reshape(_B, _NUM_LEVELS, _D)
    count = cnt[0, 0]
    return output, r_s, e_s, zh, count


# speedup vs baseline: 1.3224x; 1.3224x over previous
import jax
import jax.numpy as jnp
from jax.experimental import pallas as pl
from jax.experimental.pallas import tpu as pltpu

_NUM_LEVELS = 4
_K = 1024
_D = 256
_B = 8192
_TILE = 256
_GRID = _B // _TILE


def _vq_body(cbn_ref, x_ref, cb0_ref, cb1_ref, cb2_ref, cb3_ref,
             hi_ref, mid_ref, lo_ref,
             idx_ref, rs_ref, es_ref, zh_ref, counts_ref):
    cb_refs = (cb0_ref, cb1_ref, cb2_ref, cb3_ref)
    xl = x_ref[...]
    zh = None
    for l in range(_NUM_LEVELS):
        cb = cb_refs[l][...]
        xn = jnp.sum(xl * xl, axis=1, keepdims=True)
        xc = jax.lax.dot_general(xl, cb, (((1,), (1,)), ((), ())),
                                 preferred_element_type=jnp.float32)
        d2 = (xn - 2.0 * xc) + cbn_ref[l, :][None, :]
        d = jnp.sqrt(jnp.maximum(d2, 0.0))
        m = jnp.min(d, axis=1, keepdims=True)
        jidx = jax.lax.broadcasted_iota(jnp.int32, d.shape, 1)
        idx = jnp.min(jnp.where(d == m, jidx, _K), axis=1)
        oh = (jidx == idx[:, None]).astype(jnp.bfloat16)
        dims = (((1,), (0,)), ((), ()))
        q = (jax.lax.dot_general(oh, hi_ref[l], dims,
                                 preferred_element_type=jnp.float32)
             + jax.lax.dot_general(oh, mid_ref[l], dims,
                                   preferred_element_type=jnp.float32)
             ) + jax.lax.dot_general(oh, lo_ref[l], dims,
                                     preferred_element_type=jnp.float32)
        idx_ref[l, :] = idx
        rs_ref[:, l * _D:(l + 1) * _D] = xl
        es_ref[:, l * _D:(l + 1) * _D] = q
        counts_ref[0, l, :] = jnp.sum(oh.astype(jnp.float32), axis=0)
        zh = q if zh is None else zh + q
        xl = xl - q
    zh_ref[...] = zh


def _count_body(p_ref, cnt_ref):
    s = jnp.sum(p_ref[...], axis=0)
    cnt_ref[0, 0] = jnp.sum((s == 0.0).astype(jnp.int32))


def kernel(x, cb0, cb1, cb2, cb3):
    cbs = (cb0, cb1, cb2, cb3)
    cbn = jnp.stack([jnp.sum(cb * cb, axis=1) for cb in cbs], axis=0)
    his, mids, los = [], [], []
    for cb in cbs:
        hi = cb.astype(jnp.bfloat16)
        r1 = cb - hi.astype(jnp.float32)
        mid = r1.astype(jnp.bfloat16)
        lo = (r1 - mid.astype(jnp.float32)).astype(jnp.bfloat16)
        his.append(hi)
        mids.append(mid)
        los.append(lo)
    hi_all = jnp.stack(his)
    mid_all = jnp.stack(mids)
    lo_all = jnp.stack(los)

    out_shapes = (
        jax.ShapeDtypeStruct((_NUM_LEVELS, _B), jnp.int32),
        jax.ShapeDtypeStruct((_B, _NUM_LEVELS * _D), jnp.float32),
        jax.ShapeDtypeStruct((_B, _NUM_LEVELS * _D), jnp.float32),
        jax.ShapeDtypeStruct((_B, _D), jnp.float32),
        jax.ShapeDtypeStruct((_GRID, _NUM_LEVELS, _K), jnp.float32),
    )
    in_specs = [
        pl.BlockSpec((_NUM_LEVELS, _K), lambda i: (0, 0)),
        pl.BlockSpec((_TILE, _D), lambda i: (i, 0)),
    ] + [pl.BlockSpec((_K, _D), lambda i: (0, 0))] * _NUM_LEVELS + [
        pl.BlockSpec((_NUM_LEVELS, _K, _D), lambda i: (0, 0, 0)),
        pl.BlockSpec((_NUM_LEVELS, _K, _D), lambda i: (0, 0, 0)),
        pl.BlockSpec((_NUM_LEVELS, _K, _D), lambda i: (0, 0, 0)),
    ]
    out_specs = [
        pl.BlockSpec((_NUM_LEVELS, _TILE), lambda i: (0, i)),
        pl.BlockSpec((_TILE, _NUM_LEVELS * _D), lambda i: (i, 0)),
        pl.BlockSpec((_TILE, _NUM_LEVELS * _D), lambda i: (i, 0)),
        pl.BlockSpec((_TILE, _D), lambda i: (i, 0)),
        pl.BlockSpec((1, _NUM_LEVELS, _K), lambda i: (i, 0, 0)),
    ]
    idx, rs, es, zh, partials = pl.pallas_call(
        _vq_body,
        grid=(_GRID,),
        in_specs=in_specs,
        out_specs=out_specs,
        out_shape=out_shapes,
        compiler_params=pltpu.CompilerParams(
            dimension_semantics=("parallel",)),
    )(cbn, x, cb0, cb1, cb2, cb3, hi_all, mid_all, lo_all)

    cnt = pl.pallas_call(
        _count_body,
        in_specs=[pl.BlockSpec((_GRID, _NUM_LEVELS, _K), lambda: (0, 0, 0))],
        out_specs=pl.BlockSpec((1, 1), lambda: (0, 0),
                               memory_space=pltpu.SMEM),
        out_shape=jax.ShapeDtypeStruct((1, 1), jnp.int32),
    )(partials)

    output = idx.T.astype(jnp.int64)
    r_s = rs.reshape(_B, _NUM_LEVELS, _D)
    e_s = es.reshape(_B, _NUM_LEVELS, _D)
    count = cnt[0, 0]
    return output, r_s, e_s, zh, count
